# baseline (speedup 1.0000x reference)
import functools

import jax
import jax.numpy as jnp
from jax._src.pallas import primitives as pl_primitives
from jax.experimental import pallas as pl
from jax.experimental.pallas import tpu as pltpu

_N = 16384
_H = 256
_TM = 1024
_TK = 4096

_BTM = 256
_CHUNK = 2048


def _build_a_kernel(bounds_ref, slin_ref, vals_ref, o_ref,
                    acc_ref, slin_sm, vals_sm, sem):
    i = pl.program_id(0)
    acc_ref[...] = jnp.zeros_like(acc_ref)
    key_iota = (jax.lax.broadcasted_iota(jnp.int32, (8, 128), 0) * 128
                + jax.lax.broadcasted_iota(jnp.int32, (8, 128), 1))
    lo = bounds_ref[i]
    hi = bounds_ref[i + 1]
    base = i * (_BTM * _N)
    a0 = lo - jax.lax.rem(lo, 128)
    span = hi - a0
    nch = jax.lax.div(span + (_CHUNK - 1), _CHUNK)

    def start_dma(c, slot):
        st = pl.multiple_of(a0 + c * _CHUNK, 128)
        pltpu.make_async_copy(slin_ref.at[pl.ds(st, _CHUNK)],
                              slin_sm.at[slot], sem.at[0, slot]).start()
        pltpu.make_async_copy(vals_ref.at[pl.ds(st, _CHUNK)],
                              vals_sm.at[slot], sem.at[1, slot]).start()

    @pl.when(nch > 0)
    def _():
        start_dma(0, 0)

    def chunk_body(c, carry):
        slot = jax.lax.rem(c, 2)

        @pl.when(c + 1 < nch)
        def _():
            start_dma(c + 1, 1 - slot)

        st = pl.multiple_of(a0 + c * _CHUNK, 128)
        pltpu.make_async_copy(slin_ref.at[pl.ds(st, _CHUNK)],
                              slin_sm.at[slot], sem.at[0, slot]).wait()
        pltpu.make_async_copy(vals_ref.at[pl.ds(st, _CHUNK)],
                              vals_sm.at[slot], sem.at[1, slot]).wait()
        jlo = jnp.maximum(lo - st, 0)
        jhi = jnp.minimum(hi - st, _CHUNK)

        def edge_body(j, carry2):
            v = slin_sm[slot, j] - base
            r = vals_sm[slot, j]
            log = _N.bit_length() - 1
            dl = jax.lax.shift_right_logical(v, log)
            s = jax.lax.bitwise_and(v, _N - 1)
            dl8 = pl.multiple_of(jax.lax.bitwise_and(dl, -8), 8)
            s128 = pl.multiple_of(jax.lax.bitwise_and(s, -128), 128)
            key = jax.lax.bitwise_or(
                jax.lax.bitwise_and(
                    jax.lax.shift_right_logical(v, log - 7), 7 * 128),
                jax.lax.bitwise_and(v, 127))
            mask = key_iota == key
            valvec = jnp.full((8, 128), r.astype(jnp.float32), jnp.float32)
            pl_primitives.store(acc_ref,
                                (pl.ds(dl8, 8), pl.ds(s128, 128)),
                                valvec, mask=mask)
            return carry2

        @pl.when(jnp.logical_and(jlo == 0, jhi == _CHUNK))
        def _():
            jax.lax.fori_loop(0, _CHUNK, edge_body, 0, unroll=32)

        @pl.when(jnp.logical_or(jlo > 0, jhi < _CHUNK))
        def _():
            jax.lax.fori_loop(jlo, jhi, edge_body, 0)
        return carry

    jax.lax.fori_loop(0, nch, chunk_body, 0)
    o_ref[...] = acc_ref[...].astype(o_ref.dtype)


def _build_a(slin_padded, vals_padded, bounds):
    grid = (_N // _BTM,)
    return pl.pallas_call(
        _build_a_kernel,
        out_shape=jax.ShapeDtypeStruct((_N, _N), jnp.bfloat16),
        grid=grid,
        in_specs=[
            pl.BlockSpec(memory_space=pltpu.MemorySpace.SMEM),
            pl.BlockSpec(memory_space=pltpu.MemorySpace.HBM),
            pl.BlockSpec(memory_space=pltpu.MemorySpace.HBM),
        ],
        out_specs=pl.BlockSpec((_BTM, _N), lambda i: (i, 0)),
        scratch_shapes=[
            pltpu.VMEM((_BTM, _N), jnp.float32),
            pltpu.SMEM((2, _CHUNK), jnp.int32),
            pltpu.SMEM((2, _CHUNK), jnp.int32),
            pltpu.SemaphoreType.DMA((2, 2)),
        ],
        compiler_params=pltpu.CompilerParams(
            dimension_semantics=("parallel",)),
    )(bounds, slin_padded, vals_padded)


def _gin_mid_kernel(a_ref, z_ref, zd_ref, w1_ref, b1_ref, w2_ref, b2_ref,
                    alpha_ref, o_ref, acc_ref):
    k = pl.program_id(1)

    @pl.when(k == 0)
    def _():
        acc_ref[...] = zd_ref[...].astype(jnp.float32)

    acc_ref[...] += jnp.dot(a_ref[...], z_ref[...],
                            preferred_element_type=jnp.float32)

    @pl.when(k == pl.num_programs(1) - 1)
    def _():
        h = jnp.dot(acc_ref[...], w1_ref[...],
                    preferred_element_type=jnp.float32) + b1_ref[...]
        h = jnp.maximum(h, 0.0)
        y = jnp.dot(h, w2_ref[...],
                    preferred_element_type=jnp.float32) + b2_ref[...]
        alpha = alpha_ref[0]
        y = jnp.where(y >= 0.0, y, alpha * y)
        o_ref[...] = y.astype(o_ref.dtype)


def _gin_last_kernel(a_ref, z_ref, zd_ref, w1_ref, b1_ref, w2_ref, b2_ref,
                     scale_ref, shift_ref, wp_ref, bp_ref,
                     alpha_ref, palpha_ref, z_out_ref, p_out_ref, acc_ref):
    k = pl.program_id(1)

    @pl.when(k == 0)
    def _():
        acc_ref[...] = zd_ref[...].astype(jnp.float32)

    acc_ref[...] += jnp.dot(a_ref[...], z_ref[...],
                            preferred_element_type=jnp.float32)

    @pl.when(k == pl.num_programs(1) - 1)
    def _():
        h = jnp.dot(acc_ref[...], w1_ref[...],
                    preferred_element_type=jnp.float32) + b1_ref[...]
        h = jnp.maximum(h, 0.0)
        y = jnp.dot(h, w2_ref[...],
                    preferred_element_type=jnp.float32) + b2_ref[...]
        alpha = alpha_ref[0]
        y = jnp.where(y >= 0.0, y, alpha * y)
        z = y * scale_ref[...] + shift_ref[...]
        z_out_ref[...] = z
        p = jnp.dot(z, wp_ref[...],
                    preferred_element_type=jnp.float32) + bp_ref[...]
        palpha = palpha_ref[0]
        p_out_ref[...] = jnp.where(p >= 0.0, p, palpha * p)


def _row(v):
    return v.reshape(1, -1).astype(jnp.float32)


def _gin_mid(a_hat, z, w1, b1, w2, b2, alpha):
    grid = (_N // _TM, _N // _TK)
    return pl.pallas_call(
        _gin_mid_kernel,
        out_shape=jax.ShapeDtypeStruct((_N, _H), jnp.bfloat16),
        grid=grid,
        in_specs=[
            pl.BlockSpec((_TM, _TK), lambda i, k: (i, k)),
            pl.BlockSpec((_TK, _H), lambda i, k: (k, 0)),
            pl.BlockSpec((_TM, _H), lambda i, k: (i, 0)),
            pl.BlockSpec((_H, _H), lambda i, k: (0, 0)),
            pl.BlockSpec((1, _H), lambda i, k: (0, 0)),
            pl.BlockSpec((_H, _H), lambda i, k: (0, 0)),
            pl.BlockSpec((1, _H), lambda i, k: (0, 0)),
            pl.BlockSpec(memory_space=pltpu.MemorySpace.SMEM),
        ],
        out_specs=pl.BlockSpec((_TM, _H), lambda i, k: (i, 0)),
        scratch_shapes=[pltpu.VMEM((_TM, _H), jnp.float32)],
        compiler_params=pltpu.CompilerParams(
            dimension_semantics=("parallel", "arbitrary")),
        cost_estimate=pl.CostEstimate(
            flops=2 * _N * _N * _H + 4 * _N * _H * _H,
            transcendentals=0,
            bytes_accessed=2 * _N * _N + 2 * 2 * _N * _H + 2 * _N * _H
                           + 8 * _H * _H),
    )(a_hat, z, z, w1, b1, w2, b2, alpha)


def _gin_last(a_hat, z, w1, b1, w2, b2, scale, shift, wp, bp, alpha, palpha):
    grid = (_N // _TM, _N // _TK)
    return pl.pallas_call(
        _gin_last_kernel,
        out_shape=(jax.ShapeDtypeStruct((_N, _H), jnp.float32),
                   jax.ShapeDtypeStruct((_N, _H), jnp.float32)),
        grid=grid,
        in_specs=[
            pl.BlockSpec((_TM, _TK), lambda i, k: (i, k)),
            pl.BlockSpec((_TK, _H), lambda i, k: (k, 0)),
            pl.BlockSpec((_TM, _H), lambda i, k: (i, 0)),
            pl.BlockSpec((_H, _H), lambda i, k: (0, 0)),
            pl.BlockSpec((1, _H), lambda i, k: (0, 0)),
            pl.BlockSpec((_H, _H), lambda i, k: (0, 0)),
            pl.BlockSpec((1, _H), lambda i, k: (0, 0)),
            pl.BlockSpec((1, _H), lambda i, k: (0, 0)),
            pl.BlockSpec((1, _H), lambda i, k: (0, 0)),
            pl.BlockSpec((_H, _H), lambda i, k: (0, 0)),
            pl.BlockSpec((1, _H), lambda i, k: (0, 0)),
            pl.BlockSpec(memory_space=pltpu.MemorySpace.SMEM),
            pl.BlockSpec(memory_space=pltpu.MemorySpace.SMEM),
        ],
        out_specs=(pl.BlockSpec((_TM, _H), lambda i, k: (i, 0)),
                   pl.BlockSpec((_TM, _H), lambda i, k: (i, 0))),
        scratch_shapes=[pltpu.VMEM((_TM, _H), jnp.float32)],
        compiler_params=pltpu.CompilerParams(
            dimension_semantics=("parallel", "arbitrary")),
        cost_estimate=pl.CostEstimate(
            flops=2 * _N * _N * _H + 6 * _N * _H * _H,
            transcendentals=0,
            bytes_accessed=2 * _N * _N + 2 * 2 * _N * _H + 8 * _N * _H
                           + 12 * _H * _H),
    )(a_hat, z, z, w1, b1, w2, b2, scale, shift, wp, bp, alpha, palpha)


def kernel(x, edge_index,
           gin0_w1, gin0_b1, gin0_w2, gin0_b2,
           gin1_w1, gin1_b1, gin1_w2, gin1_b2,
           gin2_w1, gin2_b1, gin2_w2, gin2_b2,
           proj_w, proj_b, act_alpha, proj_alpha,
           enc_bn_scale, enc_bn_shift, proj_bn_scale, proj_bn_shift):
    src, dst = edge_index[0], edge_index[1]
    ne = src.shape[0]
    slin = jnp.sort(dst.astype(jnp.int32) * _N + src.astype(jnp.int32))
    idx = jnp.arange(ne, dtype=jnp.int32)
    prev = jnp.concatenate([jnp.full((1,), -1, jnp.int32), slin[:-1]])
    is_first = slin != prev
    run_start = jax.lax.cummax(jnp.where(is_first, idx, 0))
    vals = idx - run_start + 1
    tile_edges = jnp.arange(_N // _BTM + 1, dtype=jnp.int32) * (_BTM * _N)
    bounds = jnp.searchsorted(slin, tile_edges).astype(jnp.int32)
    pad = jnp.zeros((_CHUNK,), jnp.int32)
    a_hat = _build_a(jnp.concatenate([slin, pad]),
                     jnp.concatenate([vals, pad]), bounds)

    alpha = jnp.asarray(act_alpha, jnp.float32).reshape(1)
    palpha = jnp.asarray(proj_alpha, jnp.float32).reshape(1)

    wp = proj_w * proj_bn_scale[None, :]
    bp = proj_b * proj_bn_scale + proj_bn_shift

    z = x.astype(jnp.bfloat16)
    z = _gin_mid(a_hat, z, gin0_w1, _row(gin0_b1), gin0_w2, _row(gin0_b2),
                 alpha)
    z = _gin_mid(a_hat, z, gin1_w1, _row(gin1_b1), gin1_w2, _row(gin1_b2),
                 alpha)
    z3, p = _gin_last(a_hat, z, gin2_w1, _row(gin2_b1), gin2_w2,
                      _row(gin2_b2), _row(enc_bn_scale), _row(enc_bn_shift),
                      wp, _row(bp), alpha, palpha)
    return z3, p


# speedup vs baseline: 1.1012x; 1.1012x over previous
import functools

import jax
import jax.numpy as jnp
from jax.experimental import pallas as pl
from jax.experimental.pallas import tpu as pltpu

_N = 16384
_H = 256
_TM = 1024
_TK = 4096

def _gin_mid_kernel(a_ref, z_ref, zd_ref, w1_ref, b1_ref, w2_ref, b2_ref,
                    alpha_ref, o_ref, acc_ref):
    k = pl.program_id(1)

    @pl.when(k == 0)
    def _():
        acc_ref[...] = zd_ref[...].astype(jnp.float32)

    acc_ref[...] += jnp.dot(a_ref[...], z_ref[...],
                            preferred_element_type=jnp.float32)

    @pl.when(k == pl.num_programs(1) - 1)
    def _():
        h = jnp.dot(acc_ref[...], w1_ref[...],
                    preferred_element_type=jnp.float32) + b1_ref[...]
        h = jnp.maximum(h, 0.0)
        y = jnp.dot(h, w2_ref[...],
                    preferred_element_type=jnp.float32) + b2_ref[...]
        alpha = alpha_ref[0]
        y = jnp.where(y >= 0.0, y, alpha * y)
        o_ref[...] = y.astype(o_ref.dtype)


def _gin_last_kernel(a_ref, z_ref, zd_ref, w1_ref, b1_ref, w2_ref, b2_ref,
                     scale_ref, shift_ref, wp_ref, bp_ref,
                     alpha_ref, palpha_ref, z_out_ref, p_out_ref, acc_ref):
    k = pl.program_id(1)

    @pl.when(k == 0)
    def _():
        acc_ref[...] = zd_ref[...].astype(jnp.float32)

    acc_ref[...] += jnp.dot(a_ref[...], z_ref[...],
                            preferred_element_type=jnp.float32)

    @pl.when(k == pl.num_programs(1) - 1)
    def _():
        h = jnp.dot(acc_ref[...], w1_ref[...],
                    preferred_element_type=jnp.float32) + b1_ref[...]
        h = jnp.maximum(h, 0.0)
        y = jnp.dot(h, w2_ref[...],
                    preferred_element_type=jnp.float32) + b2_ref[...]
        alpha = alpha_ref[0]
        y = jnp.where(y >= 0.0, y, alpha * y)
        z = y * scale_ref[...] + shift_ref[...]
        z_out_ref[...] = z
        p = jnp.dot(z, wp_ref[...],
                    preferred_element_type=jnp.float32) + bp_ref[...]
        palpha = palpha_ref[0]
        p_out_ref[...] = jnp.where(p >= 0.0, p, palpha * p)


def _row(v):
    return v.reshape(1, -1).astype(jnp.float32)


def _gin_mid(a_hat, z, w1, b1, w2, b2, alpha):
    grid = (_N // _TM, _N // _TK)
    return pl.pallas_call(
        _gin_mid_kernel,
        out_shape=jax.ShapeDtypeStruct((_N, _H), jnp.bfloat16),
        grid=grid,
        in_specs=[
            pl.BlockSpec((_TM, _TK), lambda i, k: (i, k)),
            pl.BlockSpec((_TK, _H), lambda i, k: (k, 0)),
            pl.BlockSpec((_TM, _H), lambda i, k: (i, 0)),
            pl.BlockSpec((_H, _H), lambda i, k: (0, 0)),
            pl.BlockSpec((1, _H), lambda i, k: (0, 0)),
            pl.BlockSpec((_H, _H), lambda i, k: (0, 0)),
            pl.BlockSpec((1, _H), lambda i, k: (0, 0)),
            pl.BlockSpec(memory_space=pltpu.MemorySpace.SMEM),
        ],
        out_specs=pl.BlockSpec((_TM, _H), lambda i, k: (i, 0)),
        scratch_shapes=[pltpu.VMEM((_TM, _H), jnp.float32)],
        compiler_params=pltpu.CompilerParams(
            dimension_semantics=("parallel", "arbitrary")),
        cost_estimate=pl.CostEstimate(
            flops=2 * _N * _N * _H + 4 * _N * _H * _H,
            transcendentals=0,
            bytes_accessed=2 * _N * _N + 2 * 2 * _N * _H + 2 * _N * _H
                           + 8 * _H * _H),
    )(a_hat, z, z, w1, b1, w2, b2, alpha)


def _gin_last(a_hat, z, w1, b1, w2, b2, scale, shift, wp, bp, alpha, palpha):
    grid = (_N // _TM, _N // _TK)
    return pl.pallas_call(
        _gin_last_kernel,
        out_shape=(jax.ShapeDtypeStruct((_N, _H), jnp.float32),
                   jax.ShapeDtypeStruct((_N, _H), jnp.float32)),
        grid=grid,
        in_specs=[
            pl.BlockSpec((_TM, _TK), lambda i, k: (i, k)),
            pl.BlockSpec((_TK, _H), lambda i, k: (k, 0)),
            pl.BlockSpec((_TM, _H), lambda i, k: (i, 0)),
            pl.BlockSpec((_H, _H), lambda i, k: (0, 0)),
            pl.BlockSpec((1, _H), lambda i, k: (0, 0)),
            pl.BlockSpec((_H, _H), lambda i, k: (0, 0)),
            pl.BlockSpec((1, _H), lambda i, k: (0, 0)),
            pl.BlockSpec((1, _H), lambda i, k: (0, 0)),
            pl.BlockSpec((1, _H), lambda i, k: (0, 0)),
            pl.BlockSpec((_H, _H), lambda i, k: (0, 0)),
            pl.BlockSpec((1, _H), lambda i, k: (0, 0)),
            pl.BlockSpec(memory_space=pltpu.MemorySpace.SMEM),
            pl.BlockSpec(memory_space=pltpu.MemorySpace.SMEM),
        ],
        out_specs=(pl.BlockSpec((_TM, _H), lambda i, k: (i, 0)),
                   pl.BlockSpec((_TM, _H), lambda i, k: (i, 0))),
        scratch_shapes=[pltpu.VMEM((_TM, _H), jnp.float32)],
        compiler_params=pltpu.CompilerParams(
            dimension_semantics=("parallel", "arbitrary")),
        cost_estimate=pl.CostEstimate(
            flops=2 * _N * _N * _H + 6 * _N * _H * _H,
            transcendentals=0,
            bytes_accessed=2 * _N * _N + 2 * 2 * _N * _H + 8 * _N * _H
                           + 12 * _H * _H),
    )(a_hat, z, z, w1, b1, w2, b2, scale, shift, wp, bp, alpha, palpha)


def kernel(x, edge_index,
           gin0_w1, gin0_b1, gin0_w2, gin0_b2,
           gin1_w1, gin1_b1, gin1_w2, gin1_b2,
           gin2_w1, gin2_b1, gin2_w2, gin2_b2,
           proj_w, proj_b, act_alpha, proj_alpha,
           enc_bn_scale, enc_bn_shift, proj_bn_scale, proj_bn_shift):
    src, dst = edge_index[0], edge_index[1]
    lin = dst.astype(jnp.int32) * _N + src.astype(jnp.int32)
    a_hat = jnp.zeros((_N * _N,), jnp.bfloat16).at[lin].add(
        jnp.ones(lin.shape, jnp.bfloat16)).reshape(_N, _N)

    alpha = jnp.asarray(act_alpha, jnp.float32).reshape(1)
    palpha = jnp.asarray(proj_alpha, jnp.float32).reshape(1)

    wp = proj_w * proj_bn_scale[None, :]
    bp = proj_b * proj_bn_scale + proj_bn_shift

    z = x.astype(jnp.bfloat16)
    z = _gin_mid(a_hat, z, gin0_w1, _row(gin0_b1), gin0_w2, _row(gin0_b2),
                 alpha)
    z = _gin_mid(a_hat, z, gin1_w1, _row(gin1_b1), gin1_w2, _row(gin1_b2),
                 alpha)
    z3, p = _gin_last(a_hat, z, gin2_w1, _row(gin2_b1), gin2_w2,
                      _row(gin2_b2), _row(enc_bn_scale), _row(enc_bn_shift),
                      wp, _row(bp), alpha, palpha)
    return z3, p
